# Initial kernel scaffold; baseline (speedup 1.0000x reference)
import jax
import jax.numpy as jnp
from jax.experimental import pallas as pl


def kernel(pre_q, codebooks):
    raise NotImplementedError("write your pallas kernel here")


# speedup vs baseline: 1.3766x; 1.3766x over previous
import functools

import jax
import jax.numpy as jnp
from jax import lax
from jax.experimental import pallas as pl
from jax.experimental.pallas import tpu as pltpu
from jax.experimental.pallas import tpu_sc as plsc

COMMITMENT_COST = 0.25
EPS = 1e-8
BT = 512


def _normalize(x):
    ss = jnp.sum(x * x, axis=-1, keepdims=True)
    norm = ss * lax.rsqrt(ss)
    norm = jnp.where(ss == 0.0, 0.0, norm)
    return x * pl.reciprocal(norm + EPS, approx=True)


def _argmax_body(z_ref, cb_ref, idx_ref, idxflat_ref):
    g = pl.program_id(0)
    K = cb_ref.shape[1]
    z = z_ref[0]
    cb = cb_ref[0]
    zn = _normalize(z)
    cn = _normalize(cb)
    sim = lax.dot_general(zn, cn, (((1,), (1,)), ((), ())),
                          preferred_element_type=jnp.float32)
    sb = sim.astype(jnp.bfloat16).astype(jnp.float32)
    m = jnp.max(sb, axis=1, keepdims=True)
    ks = lax.broadcasted_iota(jnp.int32, sim.shape, 1)
    k_up = jnp.max(jnp.where(sim > m, ks, -1), axis=1)
    k_fc = jnp.min(jnp.where(sb == m, ks, K), axis=1)
    idx = jnp.where(k_up >= 0, k_up, k_fc)
    idx_ref[0, 0, :] = idx
    idxflat_ref[0, 0, :] = idx + g * K


def _run_argmax(z, codebooks):
    G, B, D = z.shape
    K = codebooks.shape[1]
    grid = (G, B // BT)
    return pl.pallas_call(
        _argmax_body,
        grid=grid,
        in_specs=[
            pl.BlockSpec((1, BT, D), lambda g, b: (g, b, 0)),
            pl.BlockSpec((1, K, D), lambda g, b: (g, 0, 0)),
        ],
        out_specs=[
            pl.BlockSpec((1, 1, BT), lambda g, b: (g, 0, b)),
            pl.BlockSpec((1, 1, BT), lambda g, b: (g, 0, b)),
        ],
        out_shape=[
            jax.ShapeDtypeStruct((G, 1, B), jnp.int32),
            jax.ShapeDtypeStruct((G, 1, B), jnp.int32),
        ],
    )(z, codebooks)


def _make_sc_gather(GB, GK, DP):
    info = plsc.get_sparse_core_info()
    NC, NS = info.num_cores, info.num_subcores
    NW = NC * NS
    chunk = GB // NW
    sub = chunk // 2
    cslice = GK // NS

    mesh = plsc.VectorSubcoreMesh(core_axis_name="c", subcore_axis_name="s")

    @functools.partial(
        pl.kernel,
        mesh=mesh,
        out_type=[
            jax.ShapeDtypeStruct((GB, DP), jnp.float32),
            jax.ShapeDtypeStruct((NC, GK), jnp.float32),
        ],
        scratch_types=[
            pltpu.VMEM((chunk,), jnp.int32),
            pltpu.VMEM((sub, DP), jnp.float32),
            pltpu.VMEM((chunk,), jnp.float32),
            pltpu.VMEM((cslice,), jnp.float32),
            pltpu.VMEM_SHARED((GK,), jnp.float32),
            pltpu.SemaphoreType.DMA,
        ],
    )
    def sc_gather(table_hbm, idxflat_hbm, q_hbm, counts_hbm,
                  idx_v, rows_v, ones_v, zbuf, shared_counts, sem):
        c = lax.axis_index("c")
        s = lax.axis_index("s")
        wid = s * NC + c
        base = wid * chunk

        def _fill_z(i, _):
            zbuf[pl.ds(i * 16, 16)] = jnp.zeros((16,), jnp.float32)
            return 0
        lax.fori_loop(0, cslice // 16, _fill_z, 0)

        def _fill_o(i, _):
            ones_v[pl.ds(i * 16, 16)] = jnp.full((16,), 1.0, jnp.float32)
            return 0
        lax.fori_loop(0, chunk // 16, _fill_o, 0)

        pltpu.sync_copy(zbuf, shared_counts.at[pl.ds(s * cslice, cslice)])

        pltpu.sync_copy(idxflat_hbm.at[pl.ds(base, chunk)], idx_v)
        for h in range(2):
            pltpu.async_copy(table_hbm.at[idx_v.at[pl.ds(h * sub, sub)]],
                             rows_v, sem).wait()
            pltpu.sync_copy(rows_v, q_hbm.at[pl.ds(base + h * sub, sub)])

        plsc.subcore_barrier()
        pltpu.sync_copy(ones_v, shared_counts.at[idx_v], add=True)
        plsc.subcore_barrier()

        pltpu.sync_copy(shared_counts.at[pl.ds(s * cslice, cslice)],
                        counts_hbm.at[c, pl.ds(s * cslice, cslice)])

    return sc_gather


def _epilogue_body(q_ref, z_ref, cnt_ref, loss_ref, perp_ref):
    g = pl.program_id(0)
    B = q_ref.shape[1]
    D = q_ref.shape[2]
    d = q_ref[0] - z_ref[0]
    ssq = jnp.sum(d * d)
    contrib = (1.0 + COMMITMENT_COST) * ssq / (B * D)

    @pl.when(g == 0)
    def _():
        loss_ref[:, :] = jnp.zeros((1, 1), jnp.float32)
    loss_ref[:, :] += jnp.full((1, 1), contrib, jnp.float32)

    cnt = cnt_ref[0, 0] + cnt_ref[0, 1]
    probs = cnt * (1.0 / B)
    ent = jnp.sum(probs * jnp.log(probs + 1e-10))
    perp_ref[0, 0, :] = jnp.full((128,), jnp.exp(-ent), jnp.float32)


def _run_epilogue(q, z, counts):
    G, B, D = q.shape
    K = counts.shape[2]
    return pl.pallas_call(
        _epilogue_body,
        grid=(G,),
        in_specs=[
            pl.BlockSpec((1, B, D), lambda g: (g, 0, 0)),
            pl.BlockSpec((1, B, D), lambda g: (g, 0, 0)),
            pl.BlockSpec((1, 2, K), lambda g: (g, 0, 0)),
        ],
        out_specs=[
            pl.BlockSpec((1, 1), lambda g: (0, 0)),
            pl.BlockSpec((1, 1, 128), lambda g: (g, 0, 0)),
        ],
        out_shape=[
            jax.ShapeDtypeStruct((1, 1), jnp.float32),
            jax.ShapeDtypeStruct((G, 1, 128), jnp.float32),
        ],
    )(q, z, counts)


def kernel(pre_q, codebooks):
    B, G, D = pre_q.shape
    K = codebooks.shape[1]
    z = jnp.transpose(pre_q, (1, 0, 2))

    idx_raw, idx_flat = _run_argmax(z, codebooks)

    DP = 128
    table = jnp.pad(codebooks.reshape(G * K, D), ((0, 0), (0, DP - D)))
    sc_gather = _make_sc_gather(G * B, G * K, DP)
    q_pad, counts2 = sc_gather(table, idx_flat.reshape(G * B))
    q_flat = q_pad[:, :D]

    counts = jnp.transpose(counts2.reshape(2, G, K), (1, 0, 2))
    loss2d, perp3d = _run_epilogue(q_flat.reshape(G, B, D), z, counts)

    q = jnp.transpose(q_flat.reshape(G, B, D), (1, 0, 2))
    indices = jnp.transpose(idx_raw[:, 0, :], (1, 0))
    return q, loss2d[0, 0], perp3d[:, 0, 0], indices


# speedup vs baseline: 1.9707x; 1.4316x over previous
import functools

import jax
import jax.numpy as jnp
from jax import lax
from jax.experimental import pallas as pl
from jax.experimental.pallas import tpu as pltpu
from jax.experimental.pallas import tpu_sc as plsc

COMMITMENT_COST = 0.25
EPS = 1e-8
BT = 512


def _normalize(x):
    ss = jnp.sum(x * x, axis=-1, keepdims=True)
    norm = ss * lax.rsqrt(ss)
    norm = jnp.where(ss == 0.0, 0.0, norm)
    return x * pl.reciprocal(norm + EPS, approx=True)


def _argmax_body(z_ref, cb_ref, idx_ref, idxflat_ref):
    g = pl.program_id(0)
    K = cb_ref.shape[1]
    z = z_ref[0]
    cb = cb_ref[0]
    zn = _normalize(z).astype(jnp.bfloat16)
    cn = _normalize(cb).astype(jnp.bfloat16)
    sim = lax.dot_general(zn, cn, (((1,), (1,)), ((), ())),
                          preferred_element_type=jnp.float32)
    m = jnp.max(sim, axis=1, keepdims=True)
    ks = lax.broadcasted_iota(jnp.int32, sim.shape, 1)
    idx = jnp.min(jnp.where(sim == m, ks, K), axis=1)
    idx_ref[0, 0, :] = idx
    idxflat_ref[0, 0, :] = idx + g * K


def _run_argmax(z, codebooks):
    G, B, D = z.shape
    K = codebooks.shape[1]
    grid = (G, B // BT)
    return pl.pallas_call(
        _argmax_body,
        grid=grid,
        in_specs=[
            pl.BlockSpec((1, BT, D), lambda g, b: (g, b, 0)),
            pl.BlockSpec((1, K, D), lambda g, b: (g, 0, 0)),
        ],
        out_specs=[
            pl.BlockSpec((1, 1, BT), lambda g, b: (g, 0, b)),
            pl.BlockSpec((1, 1, BT), lambda g, b: (g, 0, b)),
        ],
        out_shape=[
            jax.ShapeDtypeStruct((G, 1, B), jnp.int32),
            jax.ShapeDtypeStruct((G, 1, B), jnp.int32),
        ],
    )(z, codebooks)


def _make_sc_gather(GB, GK, DP):
    info = plsc.get_sparse_core_info()
    NC, NS = info.num_cores, info.num_subcores
    NW = NC * NS
    chunk = GB // NW
    sub = chunk // 2
    cslice = GK // NS

    mesh = plsc.VectorSubcoreMesh(core_axis_name="c", subcore_axis_name="s")

    @functools.partial(
        pl.kernel,
        mesh=mesh,
        out_type=[
            jax.ShapeDtypeStruct((GB, DP), jnp.float32),
            jax.ShapeDtypeStruct((NC, GK), jnp.float32),
        ],
        scratch_types=[
            pltpu.VMEM((chunk,), jnp.int32),
            pltpu.VMEM((sub, DP), jnp.float32),
            pltpu.VMEM((chunk,), jnp.float32),
            pltpu.VMEM((cslice,), jnp.float32),
            pltpu.VMEM_SHARED((GK,), jnp.float32),
            pltpu.SemaphoreType.DMA,
        ],
    )
    def sc_gather(table_hbm, idxflat_hbm, q_hbm, counts_hbm,
                  idx_v, rows_v, ones_v, zbuf, shared_counts, sem):
        c = lax.axis_index("c")
        s = lax.axis_index("s")
        wid = s * NC + c
        base = wid * chunk

        def _fill_z(i, _):
            zbuf[pl.ds(i * 16, 16)] = jnp.zeros((16,), jnp.float32)
            return 0
        lax.fori_loop(0, cslice // 16, _fill_z, 0)

        def _fill_o(i, _):
            ones_v[pl.ds(i * 16, 16)] = jnp.full((16,), 1.0, jnp.float32)
            return 0
        lax.fori_loop(0, chunk // 16, _fill_o, 0)

        pltpu.sync_copy(zbuf, shared_counts.at[pl.ds(s * cslice, cslice)])

        pltpu.sync_copy(idxflat_hbm.at[pl.ds(base, chunk)], idx_v)
        for h in range(2):
            pltpu.async_copy(table_hbm.at[idx_v.at[pl.ds(h * sub, sub)]],
                             rows_v, sem).wait()
            pltpu.sync_copy(rows_v, q_hbm.at[pl.ds(base + h * sub, sub)])

        plsc.subcore_barrier()
        pltpu.sync_copy(ones_v, shared_counts.at[idx_v], add=True)
        plsc.subcore_barrier()

        pltpu.sync_copy(shared_counts.at[pl.ds(s * cslice, cslice)],
                        counts_hbm.at[c, pl.ds(s * cslice, cslice)])

    return sc_gather


def _epilogue_body(q_ref, z_ref, cnt_ref, loss_ref, perp_ref):
    g = pl.program_id(0)
    B = q_ref.shape[1]
    D = q_ref.shape[2]
    d = q_ref[0] - z_ref[0]
    ssq = jnp.sum(d * d)
    contrib = (1.0 + COMMITMENT_COST) * ssq / (B * D)

    @pl.when(g == 0)
    def _():
        loss_ref[:, :] = jnp.zeros((1, 1), jnp.float32)
    loss_ref[:, :] += jnp.full((1, 1), contrib, jnp.float32)

    cnt = cnt_ref[0, 0] + cnt_ref[0, 1]
    probs = cnt * (1.0 / B)
    ent = jnp.sum(probs * jnp.log(probs + 1e-10))
    perp_ref[0, 0, :] = jnp.full((128,), jnp.exp(-ent), jnp.float32)


def _run_epilogue(q, z, counts):
    G, B, D = q.shape
    K = counts.shape[2]
    return pl.pallas_call(
        _epilogue_body,
        grid=(G,),
        in_specs=[
            pl.BlockSpec((1, B, D), lambda g: (g, 0, 0)),
            pl.BlockSpec((1, B, D), lambda g: (g, 0, 0)),
            pl.BlockSpec((1, 2, K), lambda g: (g, 0, 0)),
        ],
        out_specs=[
            pl.BlockSpec((1, 1), lambda g: (0, 0)),
            pl.BlockSpec((1, 1, 128), lambda g: (g, 0, 0)),
        ],
        out_shape=[
            jax.ShapeDtypeStruct((1, 1), jnp.float32),
            jax.ShapeDtypeStruct((G, 1, 128), jnp.float32),
        ],
    )(q, z, counts)


def kernel(pre_q, codebooks):
    B, G, D = pre_q.shape
    K = codebooks.shape[1]
    z = jnp.transpose(pre_q, (1, 0, 2))

    idx_raw, idx_flat = _run_argmax(z, codebooks)

    DP = 128
    table = jnp.pad(codebooks.reshape(G * K, D), ((0, 0), (0, DP - D)))
    sc_gather = _make_sc_gather(G * B, G * K, DP)
    q_pad, counts2 = sc_gather(table, idx_flat.reshape(G * B))
    q_flat = q_pad[:, :D]

    counts = jnp.transpose(counts2.reshape(2, G, K), (1, 0, 2))
    loss2d, perp3d = _run_epilogue(q_flat.reshape(G, B, D), z, counts)

    q = jnp.transpose(q_flat.reshape(G, B, D), (1, 0, 2))
    indices = jnp.transpose(idx_raw[:, 0, :], (1, 0))
    return q, loss2d[0, 0], perp3d[:, 0, 0], indices
